# Initial kernel scaffold; baseline (speedup 1.0000x reference)
import jax
import jax.numpy as jnp
from jax.experimental import pallas as pl


def kernel(x_cont, x_cat, edge_index, batch, E_charge, E_pdg, E_pv, W_cont, b_cont, W_cat, b_cat, W_enc, b_enc, g1, b1, W_msg, b_msg, g2, b2, W_o1, b_o1, W_o2, b_o2):
    raise NotImplementedError("write your pallas kernel here")


# speedup vs baseline: 1.5149x; 1.5149x over previous
import functools
import jax
import jax.numpy as jnp
from jax.experimental import pallas as pl
from jax.experimental.pallas import tpu as pltpu

N = 100000
H = 32
BLK = 5000


def _elu(x):
    return jnp.where(x > 0, x, jnp.exp(jnp.minimum(x, 0.0)) - 1.0)


def _sel_rows(idx, table, nrows):
    out = jnp.broadcast_to(table[0:1, :], (idx.shape[0], table.shape[1]))
    for r in range(1, nrows):
        out = jnp.where(idx == r, table[r:r + 1, :], out)
    return out


def _s1_body(xc_ref, cat_ref, tc_ref, tp_ref, tv_ref, wcont_ref, wenc_ref,
             bias_ref, y_ref, stat_ref):
    i = pl.program_id(0)
    xc = xc_ref[...]
    cat = cat_ref[...]
    chrg = cat[:, 1:2] + 1
    pv = cat[:, 2:3]
    pdg = jnp.abs(cat[:, 0:1])
    for k, pdgval in enumerate([1, 2, 11, 13, 22, 130, 211]):
        pdg = jnp.where(pdg == pdgval, k, pdg)
    pre_cat = (_sel_rows(chrg, tc_ref[...], 3)
               + _sel_rows(pdg, tp_ref[...], 7)
               + _sel_rows(pv, tv_ref[...], 8))
    b_cat = bias_ref[0:1, 0:16]
    b_cont = bias_ref[1:2, 0:16]
    b_enc = bias_ref[2:3, :]
    emb_cat = _elu(pre_cat + b_cat)
    emb_cont = _elu(jnp.dot(xc, wcont_ref[...],
                            preferred_element_type=jnp.float32) + b_cont)
    catted = jnp.concatenate([emb_cat, emb_cont], axis=1)
    y = _elu(jnp.dot(catted, wenc_ref[...],
                     preferred_element_type=jnp.float32) + b_enc)
    y_ref[...] = y
    s = jnp.sum(y, axis=0, keepdims=True)
    s2 = jnp.sum(y * y, axis=0, keepdims=True)
    part = jnp.concatenate([s, s2, jnp.zeros((6, H), jnp.float32)], axis=0)

    @pl.when(i == 0)
    def _():
        stat_ref[...] = part

    @pl.when(i > 0)
    def _():
        stat_ref[...] += part


def _s2_body(y_ref, st1_ref, w2_ref, bm_ref):
    s1 = st1_ref[0:1, :]
    t1 = st1_ref[1:2, :]
    emb = y_ref[...] * s1 + t1
    bm_ref[...] = jnp.dot(emb, w2_ref[...], preferred_element_type=jnp.float32)


def _s3_body(y_ref, r_ref, st1_ref, w12_ref, agg_ref, stat_ref):
    i = pl.program_id(0)
    s1 = st1_ref[0:1, :]
    t1 = st1_ref[1:2, :]
    bmsg = st1_ref[2:3, :]
    emb = y_ref[...] * s1 + t1
    a = jnp.dot(emb, w12_ref[...], preferred_element_type=jnp.float32) + bmsg
    r = r_ref[...]
    has = r[:, 0:1] > -jnp.inf
    agg = jnp.where(has, a + r, 0.0)
    agg_ref[...] = agg
    s = jnp.sum(agg, axis=0, keepdims=True)
    s2 = jnp.sum(agg * agg, axis=0, keepdims=True)
    part = jnp.concatenate([s, s2, jnp.zeros((6, H), jnp.float32)], axis=0)

    @pl.when(i == 0)
    def _():
        stat_ref[...] = part

    @pl.when(i > 0)
    def _():
        stat_ref[...] += part


def _s4_body(y_ref, agg_ref, st_ref, wo1_ref, wo2_ref, out_ref):
    s1 = st_ref[0:1, :]
    t1 = st_ref[1:2, :]
    s2 = st_ref[2:3, :]
    t2 = st_ref[3:4, :]
    b_o1 = st_ref[4:5, 0:16]
    b_o2 = st_ref[5, 16]
    emb2 = y_ref[...] * s1 + t1 + agg_ref[...] * s2 + t2
    h = _elu(jnp.dot(emb2, wo1_ref[...],
                     preferred_element_type=jnp.float32) + b_o1)
    out_ref[...] = jnp.dot(h, wo2_ref[...],
                           preferred_element_type=jnp.float32) + b_o2


def _blk(b, w):
    return pl.BlockSpec((b, w), lambda i: (i, 0))


def _full(shape):
    return pl.BlockSpec(shape, lambda i: tuple(0 for _ in shape))


def kernel(x_cont, x_cat, edge_index, batch, E_charge, E_pdg, E_pv,
           W_cont, b_cont, W_cat, b_cat, W_enc, b_enc, g1, b1,
           W_msg, b_msg, g2, b2, W_o1, b_o1, W_o2, b_o2):
    nblk = N // BLK
    WcatT = W_cat.T
    Tc = E_charge @ WcatT[0:8]
    Tp = E_pdg @ WcatT[8:16]
    Tv = E_pv @ WcatT[16:24]
    bias = jnp.zeros((8, H), jnp.float32)
    bias = bias.at[0, 0:16].set(b_cat).at[1, 0:16].set(b_cont).at[2, :].set(b_enc)
    Wt = W_msg.T
    W1t, W2t = Wt[:H], Wt[H:]
    W12 = W1t - W2t

    y, st1 = pl.pallas_call(
        _s1_body,
        grid=(nblk,),
        in_specs=[_blk(BLK, 8), _blk(BLK, 3), _full((3, 16)), _full((7, 16)),
                  _full((8, 16)), _full((8, 16)), _full((H, H)), _full((8, H))],
        out_specs=[_blk(BLK, H), _full((8, H))],
        out_shape=[jax.ShapeDtypeStruct((N, H), jnp.float32),
                   jax.ShapeDtypeStruct((8, H), jnp.float32)],
    )(x_cont, x_cat, Tc, Tp, Tv, W_cont.T, W_enc.T, bias)

    m1 = st1[0] / N
    v1 = st1[1] / N - m1 * m1
    s1 = g1 / jnp.sqrt(v1 + 1e-5)
    t1 = b1 - m1 * s1
    st1p = jnp.zeros((8, H), jnp.float32)
    st1p = st1p.at[0].set(s1).at[1].set(t1).at[2].set(b_msg)

    bm = pl.pallas_call(
        _s2_body,
        grid=(nblk,),
        in_specs=[_blk(BLK, H), _full((8, H)), _full((H, H))],
        out_specs=_blk(BLK, H),
        out_shape=jax.ShapeDtypeStruct((N, H), jnp.float32),
    )(y, st1p, W2t)

    src = edge_index[0]
    dst = edge_index[1]
    r_raw = jax.ops.segment_max(bm[src], dst, num_segments=N)

    agg, st2 = pl.pallas_call(
        _s3_body,
        grid=(nblk,),
        in_specs=[_blk(BLK, H), _blk(BLK, H), _full((8, H)), _full((H, H))],
        out_specs=[_blk(BLK, H), _full((8, H))],
        out_shape=[jax.ShapeDtypeStruct((N, H), jnp.float32),
                   jax.ShapeDtypeStruct((8, H), jnp.float32)],
    )(y, r_raw, st1p, W12)

    m2 = st2[0] / N
    v2 = st2[1] / N - m2 * m2
    s2v = g2 / jnp.sqrt(v2 + 1e-5)
    t2 = b2 - m2 * s2v
    stp = jnp.zeros((8, H), jnp.float32)
    stp = stp.at[0].set(s1).at[1].set(t1).at[2].set(s2v).at[3].set(t2)
    stp = stp.at[4, 0:16].set(b_o1).at[5, 16].set(b_o2[0])

    wo2 = jnp.zeros((16, 8), jnp.float32).at[:, 0:1].set(W_o2.T)

    out = pl.pallas_call(
        _s4_body,
        grid=(nblk,),
        in_specs=[_blk(BLK, H), _blk(BLK, H), _full((8, H)), _full((H, 16)),
                  _full((16, 8))],
        out_specs=_blk(BLK, 8),
        out_shape=jax.ShapeDtypeStruct((N, 8), jnp.float32),
    )(y, agg, stp, W_o1.T, wo2)

    return out[:, 0]
